# baseline (speedup 1.0000x reference)
import jax
import jax.numpy as jnp
import numpy as np
from jax.experimental import pallas as pl

N = 2048
DIM = 128
NC = 2
K = 512
HD = DIM // NC
BN = 2048

_HI_MASK = np.uint32(0xFFFF0000)


def _bf16_split3(v):
    hi_f = jax.lax.bitcast_convert_type(
        jax.lax.bitcast_convert_type(v, jnp.uint32) & _HI_MASK, jnp.float32)
    r1 = v - hi_f
    mid_f = jax.lax.bitcast_convert_type(
        jax.lax.bitcast_convert_type(r1, jnp.uint32) & _HI_MASK, jnp.float32)
    lo_f = r1 - mid_f
    return (hi_f.astype(jnp.bfloat16), mid_f.astype(jnp.bfloat16),
            lo_f.astype(jnp.bfloat16))


def _gather_rows(oh, ec3):
    parts = [
        jax.lax.dot_general(
            oh, t, (((1,), (0,)), ((), ())),
            preferred_element_type=jnp.float32)
        for t in ec3
    ]
    return (parts[0] + parts[1]) + parts[2]


def _vq_kernel(x_ref, embed_ref, q_ref, idx_ref):
    x = x_ref[...]
    lane_f = jax.lax.broadcasted_iota(
        jnp.int32, (BN, K), 1).astype(jnp.float32)
    lane_f2 = jax.lax.broadcasted_iota(
        jnp.int32, (2 * BN, K), 1).astype(jnp.float32)
    kf = jnp.float32(K)

    xcs = []
    ecs = []
    ds = []
    for c in range(NC):
        xc = x[:, c * HD:(c + 1) * HD]
        ec = embed_ref[c]
        ecT = jnp.transpose(ec)
        s = jax.lax.dot_general(
            xc, ecT, (((1,), (0,)), ((), ())),
            preferred_element_type=jnp.float32,
            precision=jax.lax.Precision.HIGHEST)
        en = jnp.sum(ecT * ecT, axis=0, keepdims=True)
        xcs.append(xc)
        ecs.append(ec)
        ds.append(en - 2.0 * s)

    i1s = []
    i2s = []
    for c in range(NC):
        d = ds[c]
        m1 = jnp.min(d, axis=1, keepdims=True)
        i1 = jnp.min(jnp.where(d == m1, lane_f, kf), axis=1,
                     keepdims=True)
        d2m = jnp.where(lane_f == i1, jnp.inf, d)
        m2 = jnp.min(d2m, axis=1, keepdims=True)
        i2 = jnp.min(jnp.where(d2m == m2, lane_f, kf), axis=1,
                     keepdims=True)
        i1s.append(i1)
        i2s.append(i2)

    idx_cols = []
    q_cols = []
    for c in range(NC):
        ec3 = _bf16_split3(ecs[c])
        i12 = jnp.concatenate([i1s[c], i2s[c]], axis=0)
        oh12 = (lane_f2 == i12).astype(jnp.bfloat16)
        e12 = _gather_rows(oh12, ec3)
        e1 = e12[:BN]
        e2 = e12[BN:]
        r1 = xcs[c] - e1
        r2 = xcs[c] - e2
        d1 = jnp.sum(r1 * r1, axis=1, keepdims=True)
        d2 = jnp.sum(r2 * r2, axis=1, keepdims=True)
        take2 = (d2 < d1) | ((d2 == d1) & (i2s[c] < i1s[c]))
        idx_cols.append(jnp.where(take2, i2s[c], i1s[c]).astype(jnp.int32))
        q_cols.append(jnp.where(take2, e2, e1))
    q_ref[...] = jnp.concatenate(q_cols, axis=1)
    idx_ref[...] = jnp.concatenate(idx_cols, axis=1)


@jax.jit
def kernel(x, node_type, embed):
    del node_type
    grid = (N // BN,)
    q, idx = pl.pallas_call(
        _vq_kernel,
        grid=grid,
        in_specs=[
            pl.BlockSpec((BN, DIM), lambda i: (i, 0)),
            pl.BlockSpec((NC, K, HD), lambda i: (0, 0, 0)),
        ],
        out_specs=[
            pl.BlockSpec((BN, DIM), lambda i: (i, 0)),
            pl.BlockSpec((BN, NC), lambda i: (i, 0)),
        ],
        out_shape=[
            jax.ShapeDtypeStruct((N, DIM), jnp.float32),
            jax.ShapeDtypeStruct((N, NC), jnp.int32),
        ],
    )(x, embed)
    return (q, idx, 0)


# speedup vs baseline: 1.0165x; 1.0165x over previous
import jax
import jax.numpy as jnp
import numpy as np
from jax.experimental import pallas as pl

N = 2048
DIM = 128
NC = 2
K = 512
HD = DIM // NC
BN = 1024

_HI_MASK = np.uint32(0xFFFF0000)


def _bf16_split3(v):
    hi_f = jax.lax.bitcast_convert_type(
        jax.lax.bitcast_convert_type(v, jnp.uint32) & _HI_MASK, jnp.float32)
    r1 = v - hi_f
    mid_f = jax.lax.bitcast_convert_type(
        jax.lax.bitcast_convert_type(r1, jnp.uint32) & _HI_MASK, jnp.float32)
    lo_f = r1 - mid_f
    return (hi_f.astype(jnp.bfloat16), mid_f.astype(jnp.bfloat16),
            lo_f.astype(jnp.bfloat16))


def _gather_rows(oh, ec3):
    parts = [
        jax.lax.dot_general(
            oh, t, (((1,), (0,)), ((), ())),
            preferred_element_type=jnp.float32)
        for t in ec3
    ]
    return (parts[0] + parts[1]) + parts[2]


def _vq_kernel(x_ref, embed_ref, q_ref, idx_ref):
    x = x_ref[...]
    lane_f = jax.lax.broadcasted_iota(
        jnp.int32, (BN, K), 1).astype(jnp.float32)
    lane_f2 = jax.lax.broadcasted_iota(
        jnp.int32, (2 * BN, K), 1).astype(jnp.float32)
    kf = jnp.float32(K)

    xcs = []
    ecs = []
    ds = []
    for c in range(NC):
        xc = x[:, c * HD:(c + 1) * HD]
        ec = embed_ref[c]
        ecT = jnp.transpose(ec)
        s = jax.lax.dot_general(
            xc, ecT, (((1,), (0,)), ((), ())),
            preferred_element_type=jnp.float32,
            precision=jax.lax.Precision.HIGHEST)
        en = jnp.sum(ecT * ecT, axis=0, keepdims=True)
        xcs.append(xc)
        ecs.append(ec)
        ds.append(en - 2.0 * s)

    i1s = []
    i2s = []
    for c in range(NC):
        d = ds[c]
        m1 = jnp.min(d, axis=1, keepdims=True)
        i1 = jnp.min(jnp.where(d == m1, lane_f, kf), axis=1,
                     keepdims=True)
        d2m = jnp.where(lane_f == i1, jnp.inf, d)
        m2 = jnp.min(d2m, axis=1, keepdims=True)
        i2 = jnp.min(jnp.where(d2m == m2, lane_f, kf), axis=1,
                     keepdims=True)
        i1s.append(i1)
        i2s.append(i2)

    idx_cols = []
    q_cols = []
    for c in range(NC):
        ec3 = _bf16_split3(ecs[c])
        i12 = jnp.concatenate([i1s[c], i2s[c]], axis=0)
        oh12 = (lane_f2 == i12).astype(jnp.bfloat16)
        e12 = _gather_rows(oh12, ec3)
        e1 = e12[:BN]
        e2 = e12[BN:]
        r1 = xcs[c] - e1
        r2 = xcs[c] - e2
        d1 = jnp.sum(r1 * r1, axis=1, keepdims=True)
        d2 = jnp.sum(r2 * r2, axis=1, keepdims=True)
        take2 = (d2 < d1) | ((d2 == d1) & (i2s[c] < i1s[c]))
        idx_cols.append(jnp.where(take2, i2s[c], i1s[c]).astype(jnp.int32))
        q_cols.append(jnp.where(take2, e2, e1))
    q_ref[...] = jnp.concatenate(q_cols, axis=1)
    idx_ref[...] = jnp.concatenate(idx_cols, axis=1)


@jax.jit
def kernel(x, node_type, embed):
    del node_type
    grid = (N // BN,)
    q, idx = pl.pallas_call(
        _vq_kernel,
        grid=grid,
        in_specs=[
            pl.BlockSpec((BN, DIM), lambda i: (i, 0)),
            pl.BlockSpec((NC, K, HD), lambda i: (0, 0, 0)),
        ],
        out_specs=[
            pl.BlockSpec((BN, DIM), lambda i: (i, 0)),
            pl.BlockSpec((BN, NC), lambda i: (i, 0)),
        ],
        out_shape=[
            jax.ShapeDtypeStruct((N, DIM), jnp.float32),
            jax.ShapeDtypeStruct((N, NC), jnp.int32),
        ],
    )(x, embed)
    return (q, idx, 0)
